# Initial kernel scaffold; baseline (speedup 1.0000x reference)
import jax
import jax.numpy as jnp
from jax.experimental import pallas as pl


def kernel(x, topk_indices, experts):
    raise NotImplementedError("write your pallas kernel here")


# speedup vs baseline: 1.1286x; 1.1286x over previous
import functools

import jax
import jax.numpy as jnp
from jax.experimental import pallas as pl
from jax.experimental.pallas import tpu as pltpu

NUM_EXPERTS = 8
BT = 256


def _dense_body(x_ref, idx_ref, w_ref, out_ref):
    x_blk = x_ref[...]
    idx_blk = idx_ref[...]
    eids = jax.lax.broadcasted_iota(jnp.int32, (1, 1, NUM_EXPERTS), 2)
    combine = (idx_blk[:, :, None] == eids).astype(jnp.float32).sum(axis=1)
    acc = jnp.zeros((x_blk.shape[0], w_ref.shape[2]), jnp.float32)
    for e in range(NUM_EXPERTS):
        y = jnp.dot(x_blk, w_ref[e], preferred_element_type=jnp.float32)
        acc = acc + combine[:, e][:, None] * y
    out_ref[...] = acc


def kernel(x, topk_indices, experts):
    b, s, d_in = x.shape
    d_out = experts.shape[2]
    T = b * s
    xt = x.reshape(T, d_in)
    idx = topk_indices.reshape(T, -1).astype(jnp.int32)
    grid = (T // BT,)
    out = pl.pallas_call(
        _dense_body,
        grid=grid,
        in_specs=[
            pl.BlockSpec((BT, d_in), lambda i: (i, 0)),
            pl.BlockSpec((BT, idx.shape[1]), lambda i: (i, 0)),
            pl.BlockSpec((NUM_EXPERTS, d_in, d_out), lambda i: (0, 0, 0)),
        ],
        out_specs=pl.BlockSpec((BT, d_out), lambda i: (i, 0)),
        out_shape=jax.ShapeDtypeStruct((T, d_out), jnp.float32),
    )(xt, idx, experts)
    return out.reshape(b, s, d_out)
